# Initial kernel scaffold; baseline (speedup 1.0000x reference)
import jax
import jax.numpy as jnp
from jax.experimental import pallas as pl


def kernel(x, edge_index, W1, b1, W2, b2):
    raise NotImplementedError("write your pallas kernel here")


# speedup vs baseline: 21.1002x; 21.1002x over previous
import functools

import jax
import jax.numpy as jnp
from jax import lax
from jax.experimental import pallas as pl
from jax.experimental.pallas import tpu as pltpu
from jax.experimental.pallas import tpu_sc as plsc

N = 10000
E = 320000
D_IN = 128
D_H1 = 128
D_H2 = 64

NC = 2
NS = 16
NW = NC * NS
CH = 128
NCHW = 80
EP = NW * NCHW * CH
ROWS2D = EP // CH
NPAD = 10240
PER_TILE = NPAD // NS
N_SINK = 64

_MESH = dict(core_axis_name="c", subcore_axis_name="s", num_cores=NC,
             num_subcores=NS)


@functools.partial(
    pl.kernel,
    out_type=jax.ShapeDtypeStruct((NC, NPAD), jnp.float32),
    mesh=plsc.VectorSubcoreMesh(**_MESH),
    scratch_types=[
        pltpu.VMEM((NCHW, CH), jnp.int32),
        pltpu.VMEM((CH,), jnp.float32),
        pltpu.VMEM_SHARED((NPAD,), jnp.float32),
    ],
)
def _sc_degree(dst_hbm, ones_hbm, zv_hbm, out_hbm, idx_v, ones_v, dacc):
    c = lax.axis_index("c")
    s = lax.axis_index("s")
    wid = s * NC + c
    base = s * PER_TILE
    pltpu.sync_copy(zv_hbm, dacc.at[pl.ds(base, PER_TILE)])
    pltpu.sync_copy(dst_hbm.at[pl.ds(wid * NCHW, NCHW)], idx_v)
    pltpu.sync_copy(ones_hbm, ones_v)
    plsc.subcore_barrier()

    def body(j, carry):
        pltpu.sync_copy(ones_v, dacc.at[idx_v.at[j]], add=True)
        return carry

    lax.fori_loop(0, NCHW, body, 0)
    plsc.subcore_barrier()
    pltpu.sync_copy(dacc.at[pl.ds(base, PER_TILE)],
                    out_hbm.at[c, pl.ds(base, PER_TILE)])


def _make_scatter(D):
    @functools.partial(
        pl.kernel,
        out_type=jax.ShapeDtypeStruct((NC, NPAD, D), jnp.float32),
        mesh=plsc.VectorSubcoreMesh(**_MESH),
        scratch_types=[
            pltpu.VMEM((NCHW, CH), jnp.int32),
            pltpu.VMEM((NCHW, CH), jnp.int32),
            pltpu.VMEM((CH, D), jnp.float32),
            pltpu.VMEM_SHARED((NPAD, D), jnp.float32),
            pltpu.SemaphoreType.DMA,
        ],
    )
    def _scat(tab_hbm, src_hbm, dst_hbm, z_hbm, out_hbm,
              src_v, dst_v, rows_v, acc, sem):
        c = lax.axis_index("c")
        s = lax.axis_index("s")
        wid = s * NC + c
        base = s * PER_TILE
        for k in range(PER_TILE // CH):
            pltpu.sync_copy(z_hbm, acc.at[pl.ds(base + k * CH, CH)])
        pltpu.sync_copy(src_hbm.at[pl.ds(wid * NCHW, NCHW)], src_v)
        pltpu.sync_copy(dst_hbm.at[pl.ds(wid * NCHW, NCHW)], dst_v)
        plsc.subcore_barrier()

        def body(j, carry):
            pltpu.async_copy(tab_hbm.at[src_v.at[j]], rows_v, sem).wait()
            pltpu.sync_copy(rows_v, acc.at[dst_v.at[j]], add=True)
            return carry

        lax.fori_loop(0, NCHW, body, 0)
        plsc.subcore_barrier()
        for k in range(PER_TILE // CH):
            sl = pl.ds(base + k * CH, CH)
            pltpu.sync_copy(acc.at[sl], out_hbm.at[c, sl])

    return _scat


_sc_scatter1 = _make_scatter(D_H1)


def _tc1_body(x_ref, w_ref, d0_ref, d1_ref, dinv_ref, hh_ref):
    deg = d0_ref[...] + d1_ref[...] + 1.0
    dinv = lax.rsqrt(deg)
    dinv_ref[...] = dinv
    h = jnp.dot(x_ref[...], w_ref[...], preferred_element_type=jnp.float32)
    hh_ref[...] = h * dinv


_tc1 = pl.pallas_call(
    _tc1_body,
    out_shape=(
        jax.ShapeDtypeStruct((N, 1), jnp.float32),
        jax.ShapeDtypeStruct((N, D_H1), jnp.float32),
    ),
)


def _tc2_body(a0_ref, a1_ref, hh1_ref, dinv_ref, b1_ref, w2_ref, hh2_ref):
    s = a0_ref[...] + a1_ref[...] + hh1_ref[...]
    o1 = jnp.maximum(dinv_ref[...] * s + b1_ref[...], 0.0)
    h2 = jnp.dot(o1, w2_ref[...], preferred_element_type=jnp.float32)
    hh2_ref[...] = h2 * dinv_ref[...]


_tc2 = pl.pallas_call(
    _tc2_body,
    out_shape=jax.ShapeDtypeStruct((N, D_H1), jnp.float32),
)


def _tc3_body(a0_ref, a1_ref, hh2_ref, dinv_ref, b2_ref, out_ref):
    out_ref[...] = (dinv_ref[...] * (a0_ref[...] + a1_ref[...] + hh2_ref[...])
                    + b2_ref[...])


_tc3 = pl.pallas_call(
    _tc3_body,
    out_shape=jax.ShapeDtypeStruct((N, D_H2), jnp.float32),
)


def kernel(x, edge_index, W1, b1, W2, b2):
    src = edge_index[0].astype(jnp.int32)
    dst = edge_index[1].astype(jnp.int32)
    pad = EP - E
    ar = jnp.arange(pad, dtype=jnp.int32)
    src_p = jnp.concatenate([src, (ar * 131) % N]).reshape(ROWS2D, CH)
    dst_p = jnp.concatenate([dst, N + (ar % N_SINK)]).reshape(ROWS2D, CH)

    ones = jnp.ones((CH,), jnp.float32)
    zv = jnp.zeros((PER_TILE,), jnp.float32)
    z1 = jnp.zeros((CH, D_H1), jnp.float32)
    W2p = jnp.pad(W2, ((0, 0), (0, D_H1 - D_H2)))

    degp = _sc_degree(dst_p, ones, zv)
    d0 = degp[0, :N, None]
    d1 = degp[1, :N, None]

    dinv, hh1 = _tc1(x, W1, d0, d1)

    acc1 = _sc_scatter1(hh1, src_p, dst_p, z1)
    hh2 = _tc2(acc1[0, :N], acc1[1, :N], hh1, dinv, b1[None, :], W2p)

    acc2 = _sc_scatter1(hh2, src_p, dst_p, z1)
    out = _tc3(acc2[0, :N, :D_H2], acc2[1, :N, :D_H2], hh2[:, :D_H2],
               dinv, b2[None, :])
    return out


# speedup vs baseline: 29.4337x; 1.3949x over previous
import functools

import jax
import jax.numpy as jnp
from jax import lax
from jax.experimental import pallas as pl
from jax.experimental.pallas import tpu as pltpu
from jax.experimental.pallas import tpu_sc as plsc

N = 10000
E = 320000
D_IN = 128
D_H1 = 128
D_H2 = 64

NC = 2
NS = 16
NW = NC * NS
CH = 128
NCHW = 80
HCH = 40
EP = NW * NCHW * CH
ROWS2D = EP // CH
NPAD = 10240
PER_TILE = NPAD // NS
N_SINK = 64

_MESH = dict(core_axis_name="c", subcore_axis_name="s", num_cores=NC,
             num_subcores=NS)


@functools.partial(
    pl.kernel,
    out_type=jax.ShapeDtypeStruct((NC, NPAD), jnp.float32),
    mesh=plsc.VectorSubcoreMesh(**_MESH),
    scratch_types=[
        pltpu.VMEM((NCHW, CH), jnp.int32),
        pltpu.VMEM((CH,), jnp.float32),
        pltpu.VMEM_SHARED((NPAD,), jnp.float32),
    ],
)
def _sc_degree(dst_hbm, ones_hbm, zv_hbm, out_hbm, idx_v, ones_v, dacc):
    c = lax.axis_index("c")
    s = lax.axis_index("s")
    wid = s * NC + c
    base = s * PER_TILE
    pltpu.sync_copy(zv_hbm, dacc.at[pl.ds(base, PER_TILE)])
    pltpu.sync_copy(dst_hbm.at[pl.ds(wid * NCHW, NCHW)], idx_v)
    pltpu.sync_copy(ones_hbm, ones_v)
    plsc.subcore_barrier()

    def body(j, carry):
        pltpu.sync_copy(ones_v, dacc.at[idx_v.at[j]], add=True)
        return carry

    lax.fori_loop(0, NCHW, body, 0)
    plsc.subcore_barrier()
    pltpu.sync_copy(dacc.at[pl.ds(base, PER_TILE)],
                    out_hbm.at[c, pl.ds(base, PER_TILE)])


def _make_scatter(D):
    @functools.partial(
        pl.kernel,
        out_type=jax.ShapeDtypeStruct((NC, NPAD, D), jnp.float32),
        mesh=plsc.VectorSubcoreMesh(**_MESH),
        scratch_types=[
            pltpu.VMEM((HCH, CH), jnp.int32),
            pltpu.VMEM((HCH, CH), jnp.int32),
            pltpu.VMEM((CH, D), jnp.float32),
            pltpu.VMEM((CH, D), jnp.float32),
            pltpu.VMEM_SHARED((NPAD, D), jnp.float32),
            pltpu.SemaphoreType.DMA,
            pltpu.SemaphoreType.DMA,
        ],
    )
    def _scat(tab_hbm, src_hbm, dst_hbm, z_hbm, out_hbm,
              src_v, dst_v, rows0, rows1, acc, sem0, sem1):
        c = lax.axis_index("c")
        s = lax.axis_index("s")
        wid = s * NC + c
        base = s * PER_TILE
        for k in range(PER_TILE // CH):
            pltpu.sync_copy(z_hbm, acc.at[pl.ds(base + k * CH, CH)])

        def body(i, carry):
            j = 2 * i
            for b, rows, sem in ((0, rows0, sem0), (1, rows1, sem1)):
                jj = j + b
                pltpu.make_async_copy(tab_hbm.at[src_v.at[jj]], rows,
                                      sem).wait()
                pltpu.sync_copy(rows, acc.at[dst_v.at[jj]], add=True)

                @pl.when(jj + 2 < HCH)
                def _():
                    pltpu.async_copy(tab_hbm.at[src_v.at[jj + 2]], rows, sem)
            return carry

        for h in range(NCHW // HCH):
            hb = wid * NCHW + h * HCH
            pltpu.sync_copy(src_hbm.at[pl.ds(hb, HCH)], src_v)
            pltpu.sync_copy(dst_hbm.at[pl.ds(hb, HCH)], dst_v)
            pltpu.async_copy(tab_hbm.at[src_v.at[0]], rows0, sem0)
            pltpu.async_copy(tab_hbm.at[src_v.at[1]], rows1, sem1)
            if h == 0:
                plsc.subcore_barrier()
            lax.fori_loop(0, HCH // 2, body, 0)
        plsc.subcore_barrier()
        for k in range(PER_TILE // CH):
            sl = pl.ds(base + k * CH, CH)
            pltpu.sync_copy(acc.at[sl], out_hbm.at[c, sl])

    return _scat


_sc_scatter1 = _make_scatter(D_H1)


def _tc1_body(x_ref, w_ref, d0_ref, d1_ref, dinv_ref, hh_ref):
    deg = d0_ref[...] + d1_ref[...] + 1.0
    dinv = lax.rsqrt(deg)
    dinv_ref[...] = dinv
    h = jnp.dot(x_ref[...], w_ref[...], preferred_element_type=jnp.float32)
    hh_ref[...] = h * dinv


_tc1 = pl.pallas_call(
    _tc1_body,
    out_shape=(
        jax.ShapeDtypeStruct((N, 1), jnp.float32),
        jax.ShapeDtypeStruct((N, D_H1), jnp.float32),
    ),
)


def _tc2_body(a0_ref, a1_ref, hh1_ref, dinv_ref, b1_ref, w2_ref, hh2_ref):
    s = a0_ref[...] + a1_ref[...] + hh1_ref[...]
    o1 = jnp.maximum(dinv_ref[...] * s + b1_ref[...], 0.0)
    h2 = jnp.dot(o1, w2_ref[...], preferred_element_type=jnp.float32)
    hh2_ref[...] = h2 * dinv_ref[...]


_tc2 = pl.pallas_call(
    _tc2_body,
    out_shape=jax.ShapeDtypeStruct((N, D_H1), jnp.float32),
)


def _tc3_body(a0_ref, a1_ref, hh2_ref, dinv_ref, b2_ref, out_ref):
    out_ref[...] = (dinv_ref[...] * (a0_ref[...] + a1_ref[...] + hh2_ref[...])
                    + b2_ref[...])


_tc3 = pl.pallas_call(
    _tc3_body,
    out_shape=jax.ShapeDtypeStruct((N, D_H2), jnp.float32),
)


def kernel(x, edge_index, W1, b1, W2, b2):
    src = edge_index[0].astype(jnp.int32)
    dst = edge_index[1].astype(jnp.int32)
    pad = EP - E
    ar = jnp.arange(pad, dtype=jnp.int32)
    src_p = jnp.concatenate([src, (ar * 131) % N]).reshape(ROWS2D, CH)
    dst_p = jnp.concatenate([dst, N + (ar % N_SINK)]).reshape(ROWS2D, CH)

    ones = jnp.ones((CH,), jnp.float32)
    zv = jnp.zeros((PER_TILE,), jnp.float32)
    z1 = jnp.zeros((CH, D_H1), jnp.float32)
    W2p = jnp.pad(W2, ((0, 0), (0, D_H1 - D_H2)))

    degp = _sc_degree(dst_p, ones, zv)
    d0 = degp[0, :N, None]
    d1 = degp[1, :N, None]

    dinv, hh1 = _tc1(x, W1, d0, d1)

    acc1 = _sc_scatter1(hh1, src_p, dst_p, z1)
    hh2 = _tc2(acc1[0, :N], acc1[1, :N], hh1, dinv, b1[None, :], W2p)

    acc2 = _sc_scatter1(hh2, src_p, dst_p, z1)
    out = _tc3(acc2[0, :N, :D_H2], acc2[1, :N, :D_H2], hh2[:, :D_H2],
               dinv, b2[None, :])
    return out


# speedup vs baseline: 30.7271x; 1.0439x over previous
import functools

import jax
import jax.numpy as jnp
from jax import lax
from jax.experimental import pallas as pl
from jax.experimental.pallas import tpu as pltpu
from jax.experimental.pallas import tpu_sc as plsc

N = 10000
E = 320000
D_IN = 128
D_H1 = 128
D_H2 = 64

NC = 2
NS = 16
NW = NC * NS
CH = 128
NCHW = 80
HCH = 40
EP = NW * NCHW * CH
ROWS2D = EP // CH
NPAD = 10240
PER_TILE = NPAD // NS
N_SINK = 64

_MESH = dict(core_axis_name="c", subcore_axis_name="s", num_cores=NC,
             num_subcores=NS)


@functools.partial(
    pl.kernel,
    out_type=jax.ShapeDtypeStruct((NC, NPAD), jnp.float32),
    mesh=plsc.VectorSubcoreMesh(**_MESH),
    scratch_types=[
        pltpu.VMEM((NCHW, CH), jnp.int32),
        pltpu.VMEM((CH,), jnp.float32),
        pltpu.VMEM_SHARED((NPAD,), jnp.float32),
    ],
)
def _sc_degree(dst_hbm, ones_hbm, zv_hbm, out_hbm, idx_v, ones_v, dacc):
    c = lax.axis_index("c")
    s = lax.axis_index("s")
    wid = s * NC + c
    base = s * PER_TILE
    pltpu.sync_copy(zv_hbm, dacc.at[pl.ds(base, PER_TILE)])
    pltpu.sync_copy(dst_hbm.at[pl.ds(wid * NCHW, NCHW)], idx_v)
    pltpu.sync_copy(ones_hbm, ones_v)
    plsc.subcore_barrier()

    def body(j, carry):
        pltpu.sync_copy(ones_v, dacc.at[idx_v.at[j]], add=True)
        return carry

    lax.fori_loop(0, NCHW, body, 0)
    plsc.subcore_barrier()
    pltpu.sync_copy(dacc.at[pl.ds(base, PER_TILE)],
                    out_hbm.at[c, pl.ds(base, PER_TILE)])


def _make_scatter(D):
    @functools.partial(
        pl.kernel,
        out_type=jax.ShapeDtypeStruct((NC, NPAD, D), jnp.float32),
        mesh=plsc.VectorSubcoreMesh(**_MESH),
        scratch_types=[
            pltpu.VMEM((HCH, CH), jnp.int32),
            pltpu.VMEM((HCH, CH), jnp.int32),
            pltpu.VMEM((CH, D), jnp.float32),
            pltpu.VMEM((CH, D), jnp.float32),
            pltpu.VMEM_SHARED((NPAD, D), jnp.float32),
            pltpu.SemaphoreType.DMA,
            pltpu.SemaphoreType.DMA,
        ],
    )
    def _scat(tab_hbm, src_hbm, dst_hbm, z_hbm, out_hbm,
              src_v, dst_v, rows0, rows1, acc, sem0, sem1):
        c = lax.axis_index("c")
        s = lax.axis_index("s")
        wid = s * NC + c
        base = s * PER_TILE
        for k in range(PER_TILE // CH):
            pltpu.sync_copy(z_hbm, acc.at[pl.ds(base + k * CH, CH)])

        def body(i, carry):
            j = 2 * i
            for b, rows, sem in ((0, rows0, sem0), (1, rows1, sem1)):
                jj = j + b
                pltpu.make_async_copy(tab_hbm.at[src_v.at[jj]], rows,
                                      sem).wait()
                pltpu.sync_copy(rows, acc.at[dst_v.at[jj]], add=True)

                @pl.when(jj + 2 < HCH)
                def _():
                    pltpu.async_copy(tab_hbm.at[src_v.at[jj + 2]], rows, sem)
            return carry

        for h in range(NCHW // HCH):
            hb = wid * NCHW + h * HCH
            pltpu.sync_copy(src_hbm.at[pl.ds(hb, HCH)], src_v)
            pltpu.sync_copy(dst_hbm.at[pl.ds(hb, HCH)], dst_v)
            pltpu.async_copy(tab_hbm.at[src_v.at[0]], rows0, sem0)
            pltpu.async_copy(tab_hbm.at[src_v.at[1]], rows1, sem1)
            if h == 0:
                plsc.subcore_barrier()
            lax.fori_loop(0, HCH // 2, body, 0)
        plsc.subcore_barrier()
        for k in range(PER_TILE // CH):
            sl = pl.ds(base + k * CH, CH)
            pltpu.sync_copy(acc.at[sl], out_hbm.at[c, sl])

    return _scat


_sc_scatter1 = _make_scatter(D_H1)


NB = 2000
_GRID = N // NB

_full = lambda shp: pl.BlockSpec(shp, lambda i: (0,) * len(shp))


def _tc1_body(x_ref, w_ref, degp_ref, dinv_ref, hh_ref):
    deg = degp_ref[0] + degp_ref[1] + 1.0
    dinv = lax.rsqrt(deg)
    dinv_ref[...] = dinv
    h = jnp.dot(x_ref[...], w_ref[...], preferred_element_type=jnp.float32)
    hh_ref[...] = h * dinv


_tc1 = pl.pallas_call(
    _tc1_body,
    grid=(_GRID,),
    in_specs=[
        pl.BlockSpec((NB, D_IN), lambda i: (i, 0)),
        _full((D_IN, D_H1)),
        pl.BlockSpec((NC, NB, 1), lambda i: (0, i, 0)),
    ],
    out_specs=(
        pl.BlockSpec((NB, 1), lambda i: (i, 0)),
        pl.BlockSpec((NB, D_H1), lambda i: (i, 0)),
    ),
    out_shape=(
        jax.ShapeDtypeStruct((N, 1), jnp.float32),
        jax.ShapeDtypeStruct((N, D_H1), jnp.float32),
    ),
)


def _tc2_body(acc_ref, hh1_ref, dinv_ref, b1_ref, w2_ref, hh2_ref):
    s = acc_ref[0] + acc_ref[1] + hh1_ref[...]
    o1 = jnp.maximum(dinv_ref[...] * s + b1_ref[...], 0.0)
    h2 = jnp.dot(o1, w2_ref[...], preferred_element_type=jnp.float32)
    hh2_ref[...] = h2 * dinv_ref[...]


_tc2 = pl.pallas_call(
    _tc2_body,
    grid=(_GRID,),
    in_specs=[
        pl.BlockSpec((NC, NB, D_H1), lambda i: (0, i, 0)),
        pl.BlockSpec((NB, D_H1), lambda i: (i, 0)),
        pl.BlockSpec((NB, 1), lambda i: (i, 0)),
        _full((1, D_H1)),
        _full((D_H1, D_H1)),
    ],
    out_specs=pl.BlockSpec((NB, D_H1), lambda i: (i, 0)),
    out_shape=jax.ShapeDtypeStruct((N, D_H1), jnp.float32),
)


def _tc3_body(acc_ref, hh2_ref, dinv_ref, b2_ref, out_ref):
    out_ref[...] = (dinv_ref[...]
                    * (acc_ref[0, :, :D_H2] + acc_ref[1, :, :D_H2]
                       + hh2_ref[:, :D_H2])
                    + b2_ref[...])


_tc3 = pl.pallas_call(
    _tc3_body,
    grid=(_GRID,),
    in_specs=[
        pl.BlockSpec((NC, NB, D_H1), lambda i: (0, i, 0)),
        pl.BlockSpec((NB, D_H1), lambda i: (i, 0)),
        pl.BlockSpec((NB, 1), lambda i: (i, 0)),
        _full((1, D_H2)),
    ],
    out_specs=pl.BlockSpec((NB, D_H2), lambda i: (i, 0)),
    out_shape=jax.ShapeDtypeStruct((N, D_H2), jnp.float32),
)


def kernel(x, edge_index, W1, b1, W2, b2):
    src = edge_index[0].astype(jnp.int32)
    dst = edge_index[1].astype(jnp.int32)
    pad = EP - E
    ar = jnp.arange(pad, dtype=jnp.int32)
    src_p = jnp.concatenate([src, (ar * 131) % N]).reshape(ROWS2D, CH)
    dst_p = jnp.concatenate([dst, N + (ar % N_SINK)]).reshape(ROWS2D, CH)

    ones = jnp.ones((CH,), jnp.float32)
    zv = jnp.zeros((PER_TILE,), jnp.float32)
    z1 = jnp.zeros((CH, D_H1), jnp.float32)
    W2p = jnp.pad(W2, ((0, 0), (0, D_H1 - D_H2)))

    degp = _sc_degree(dst_p, ones, zv)

    dinv, hh1 = _tc1(x, W1, degp[:, :, None])

    acc1 = _sc_scatter1(hh1, src_p, dst_p, z1)
    hh2 = _tc2(acc1, hh1, dinv, b1[None, :], W2p)

    acc2 = _sc_scatter1(hh2, src_p, dst_p, z1)
    out = _tc3(acc2, hh2, dinv, b2[None, :])
    return out
